# baseline (speedup 1.0000x reference)
import functools

import numpy as np
import jax
import jax.numpy as jnp
from jax import lax
from jax.experimental import pallas as pl
from jax.experimental.pallas import tpu as pltpu
from jax.experimental.pallas import tpu_sc as plsc

_NUM_CORES = 2
_NUM_SUBCORES = 16
_LANES = 16
_NW = _NUM_CORES * _NUM_SUBCORES
_IL = 4
_SC_SLABS = 4
_TC_BM = 512


def _u_thresholds(i_dim: int, m_dim: int) -> jax.Array:
    u = jax.random.uniform(jax.random.key(42), (i_dim, m_dim, 1),
                           dtype=jnp.float32)
    return u.reshape(i_dim, m_dim)


@functools.lru_cache(maxsize=2)
def _build_sc_call(i_dim: int, m_dim: int, np1: int, sc_slabs: int):
    wps = _NW // sc_slabs
    rows_per_w = m_dim // wps
    i0 = i_dim - sc_slabs
    chunk = 128
    assert rows_per_w % chunk == 0 and chunk % (_LANES * _IL) == 0
    n_chunks = rows_per_w // chunk
    assert n_chunks % 2 == 0
    groups_per_chunk = chunk // _LANES

    mesh = plsc.VectorSubcoreMesh(core_axis_name="c", subcore_axis_name="s")

    @functools.partial(
        pl.kernel,
        out_type=jax.ShapeDtypeStruct((sc_slabs, m_dim), jnp.int32),
        mesh=mesh,
        compiler_params=pltpu.CompilerParams(needs_layout_passes=False,
                                             use_tc_tiling_on_sc=True),
        scratch_types=[
            pltpu.VMEM((chunk, np1), jnp.float32),
            pltpu.VMEM((chunk, np1), jnp.float32),
            pltpu.VMEM((rows_per_w,), jnp.float32),
            pltpu.VMEM((rows_per_w,), jnp.int32),
            pltpu.SemaphoreType.DMA,
            pltpu.SemaphoreType.DMA,
        ],
    )
    def sc_count(pi_hbm, u_hbm, out_hbm, buf_a, buf_b, u_v, out_v,
                 sem_a, sem_b):
        wid = lax.axis_index("s") * _NUM_CORES + lax.axis_index("c")
        slab = wid // wps
        row0 = (wid % wps) * rows_per_w
        pltpu.sync_copy(u_hbm.at[i0 + slab, pl.ds(row0, rows_per_w)], u_v)

        bufs = (buf_a, buf_b)
        sems = (sem_a, sem_b)

        def chunk_src(ci):
            return pi_hbm.at[i0 + slab, pl.ds(row0 + ci * chunk, chunk), :]

        pltpu.async_copy(chunk_src(0), bufs[0], sems[0])

        lane = lax.iota(jnp.int32, _LANES)

        @pl.loop(0, n_chunks, step=2)
        def _chunk_loop(ci):
            for b in range(2):
                cur = ci + b

                @pl.when(cur + 1 < n_chunks)
                def _start_next():
                    pltpu.async_copy(chunk_src(cur + 1), bufs[1 - b],
                                     sems[1 - b])

                pltpu.make_async_copy(chunk_src(cur), bufs[b], sems[b]).wait()
                buf = bufs[b]

                @plsc.parallel_loop(0, groups_per_chunk, unroll=_IL)
                def _group_loop(g):
                    out_base = cur * chunk + g * _LANES
                    u_vec = u_v[pl.ds(out_base, _LANES)]
                    rows = g * _LANES + lane
                    acc = jnp.zeros((_LANES,), jnp.float32)
                    cnt = jnp.zeros((_LANES,), jnp.int32)
                    for k in range(np1):
                        col = jnp.full((_LANES,), k, jnp.int32)
                        v = plsc.load_gather(buf, [rows, col])
                        acc = acc + v
                        cnt = jnp.where(u_vec > acc, cnt + 1, cnt)
                    out_v[pl.ds(out_base, _LANES)] = cnt

        pltpu.sync_copy(out_v, out_hbm.at[slab, pl.ds(row0, rows_per_w)])

    return sc_count


def _tc_body(pi_ref, u_ref, out_ref):
    x = pi_ref[...]
    np1 = x.shape[-1]
    row = lax.broadcasted_iota(jnp.int32, (np1, np1), 0)
    col = lax.broadcasted_iota(jnp.int32, (np1, np1), 1)
    tri = (row <= col).astype(jnp.float32)
    cum = jnp.dot(x, tri, preferred_element_type=jnp.float32)
    u = u_ref[0, 0, :]
    sel = (u[:, None] > cum).astype(jnp.float32)
    ones = jnp.ones((np1, 1), jnp.float32)
    cnt = jnp.dot(sel, ones, preferred_element_type=jnp.float32)
    out_ref[0, 0, :] = cnt[:, 0].astype(jnp.int32)


@functools.lru_cache(maxsize=2)
def _build_tc_call(tc_rows: int, np1: int):
    assert tc_rows % _TC_BM == 0
    nb = tc_rows // _TC_BM
    return pl.pallas_call(
        _tc_body,
        grid=(nb,),
        in_specs=[
            pl.BlockSpec((_TC_BM, np1), lambda j: (j, 0)),
            pl.BlockSpec((1, 1, _TC_BM), lambda j: (j, 0, 0)),
        ],
        out_specs=pl.BlockSpec((1, 1, _TC_BM), lambda j: (j, 0, 0)),
        out_shape=jax.ShapeDtypeStruct((nb, 1, _TC_BM), jnp.int32),
    )


def kernel(pi_vectors):
    i_dim, m_dim, np1 = pi_vectors.shape
    u = _u_thresholds(i_dim, m_dim)
    i_tc = i_dim - _SC_SLABS
    tc_rows = i_tc * m_dim

    pi_flat = pi_vectors.reshape(i_dim * m_dim, np1)
    u_blk = u.reshape(i_dim * m_dim // _TC_BM, 1, _TC_BM)
    tc_out = _build_tc_call(tc_rows, np1)(pi_flat, u_blk)

    sc_out = _build_sc_call(i_dim, m_dim, np1, _SC_SLABS)(pi_vectors, u)

    return jnp.concatenate(
        [tc_out.reshape(i_tc, m_dim), sc_out], axis=0)


# speedup vs baseline: 2.2145x; 2.2145x over previous
import functools

import numpy as np
import jax
import jax.numpy as jnp
from jax import lax
from jax.experimental import pallas as pl
from jax.experimental.pallas import tpu as pltpu
from jax.experimental.pallas import tpu_sc as plsc

_NUM_CORES = 2
_NUM_SUBCORES = 16
_LANES = 16
_NW = _NUM_CORES * _NUM_SUBCORES
_IL = 4
_SC_SLABS = 4
_TC_BM = 4096


def _u_thresholds(i_dim: int, m_dim: int) -> jax.Array:
    u = jax.random.uniform(jax.random.key(42), (i_dim, m_dim, 1),
                           dtype=jnp.float32)
    return u.reshape(i_dim, m_dim)


@functools.lru_cache(maxsize=2)
def _build_sc_call(i_dim: int, m_dim: int, np1: int, sc_slabs: int):
    wps = _NW // sc_slabs
    rows_per_w = m_dim // wps
    i0 = i_dim - sc_slabs
    chunk = 128
    assert rows_per_w % chunk == 0 and chunk % (_LANES * _IL) == 0
    n_chunks = rows_per_w // chunk
    assert n_chunks % 2 == 0
    groups_per_chunk = chunk // _LANES

    mesh = plsc.VectorSubcoreMesh(core_axis_name="c", subcore_axis_name="s")

    @functools.partial(
        pl.kernel,
        out_type=jax.ShapeDtypeStruct((sc_slabs, m_dim), jnp.int32),
        mesh=mesh,
        compiler_params=pltpu.CompilerParams(needs_layout_passes=False,
                                             use_tc_tiling_on_sc=True),
        scratch_types=[
            pltpu.VMEM((chunk, np1), jnp.float32),
            pltpu.VMEM((chunk, np1), jnp.float32),
            pltpu.VMEM((rows_per_w,), jnp.float32),
            pltpu.VMEM((rows_per_w,), jnp.int32),
            pltpu.SemaphoreType.DMA,
            pltpu.SemaphoreType.DMA,
        ],
    )
    def sc_count(pi_hbm, u_hbm, out_hbm, buf_a, buf_b, u_v, out_v,
                 sem_a, sem_b):
        wid = lax.axis_index("s") * _NUM_CORES + lax.axis_index("c")
        slab = wid // wps
        row0 = (wid % wps) * rows_per_w
        pltpu.sync_copy(u_hbm.at[i0 + slab, pl.ds(row0, rows_per_w)], u_v)

        bufs = (buf_a, buf_b)
        sems = (sem_a, sem_b)

        def chunk_src(ci):
            return pi_hbm.at[i0 + slab, pl.ds(row0 + ci * chunk, chunk), :]

        pltpu.async_copy(chunk_src(0), bufs[0], sems[0])

        lane = lax.iota(jnp.int32, _LANES)

        @pl.loop(0, n_chunks, step=2)
        def _chunk_loop(ci):
            for b in range(2):
                cur = ci + b

                @pl.when(cur + 1 < n_chunks)
                def _start_next():
                    pltpu.async_copy(chunk_src(cur + 1), bufs[1 - b],
                                     sems[1 - b])

                pltpu.make_async_copy(chunk_src(cur), bufs[b], sems[b]).wait()
                buf = bufs[b]

                @plsc.parallel_loop(0, groups_per_chunk, unroll=_IL)
                def _group_loop(g):
                    out_base = cur * chunk + g * _LANES
                    u_vec = u_v[pl.ds(out_base, _LANES)]
                    rows = g * _LANES + lane
                    acc = jnp.zeros((_LANES,), jnp.float32)
                    cnt = jnp.zeros((_LANES,), jnp.int32)
                    for k in range(np1):
                        col = jnp.full((_LANES,), k, jnp.int32)
                        v = plsc.load_gather(buf, [rows, col])
                        acc = acc + v
                        cnt = jnp.where(u_vec > acc, cnt + 1, cnt)
                    out_v[pl.ds(out_base, _LANES)] = cnt

        pltpu.sync_copy(out_v, out_hbm.at[slab, pl.ds(row0, rows_per_w)])

    return sc_count


def _tc_body(pi_ref, u_ref, out_ref):
    x = pi_ref[...]
    np1 = x.shape[-1]
    row = lax.broadcasted_iota(jnp.int32, (np1, np1), 0)
    col = lax.broadcasted_iota(jnp.int32, (np1, np1), 1)
    tri = (row <= col).astype(jnp.float32)
    cum = jnp.dot(x, tri, preferred_element_type=jnp.float32)
    u = u_ref[0, 0, :]
    sel = (u[:, None] > cum).astype(jnp.float32)
    ones = jnp.ones((1, np1), jnp.float32)
    cnt = lax.dot_general(ones, sel, (((1,), (1,)), ((), ())),
                          preferred_element_type=jnp.float32)
    out_ref[0, 0, :] = cnt[0, :].astype(jnp.int32)


@functools.lru_cache(maxsize=2)
def _build_tc_call(tc_rows: int, np1: int):
    assert tc_rows % _TC_BM == 0
    nb = tc_rows // _TC_BM
    return pl.pallas_call(
        _tc_body,
        grid=(nb,),
        in_specs=[
            pl.BlockSpec((_TC_BM, np1), lambda j: (j, 0)),
            pl.BlockSpec((1, 1, _TC_BM), lambda j: (j, 0, 0)),
        ],
        out_specs=pl.BlockSpec((1, 1, _TC_BM), lambda j: (j, 0, 0)),
        out_shape=jax.ShapeDtypeStruct((nb, 1, _TC_BM), jnp.int32),
    )


def kernel(pi_vectors):
    i_dim, m_dim, np1 = pi_vectors.shape
    u = _u_thresholds(i_dim, m_dim)
    i_tc = i_dim - _SC_SLABS
    tc_rows = i_tc * m_dim

    pi_flat = pi_vectors.reshape(i_dim * m_dim, np1)
    u_blk = u.reshape(i_dim * m_dim // _TC_BM, 1, _TC_BM)
    tc_out = _build_tc_call(tc_rows, np1)(pi_flat, u_blk)

    sc_out = _build_sc_call(i_dim, m_dim, np1, _SC_SLABS)(pi_vectors, u)

    return jnp.concatenate(
        [tc_out.reshape(i_tc, m_dim), sc_out], axis=0)
